# Initial kernel scaffold; baseline (speedup 1.0000x reference)
import jax
import jax.numpy as jnp
from jax.experimental import pallas as pl


def kernel(x, layer_idx, W_ih, W_hh, b_ih, b_hh, ce_w1, ce_b1, ce_w2, ce_b2, rh_w, rh_b, tk_w1, tk_b1, tk_w2, tk_b2):
    raise NotImplementedError("write your pallas kernel here")


# speedup vs baseline: 7.8455x; 7.8455x over previous
import functools

import jax
import jax.numpy as jnp
from jax import lax
from jax.experimental import pallas as pl
from jax.experimental.pallas import tpu as pltpu

B, S, D = 4, 2048, 1024
E = 8
H = 256
MAXK = 4
T = 128
NBLK = S // T

_PREC = lax.Precision.DEFAULT


def _fused_body(x_ref, wxT_ref, whhT_ref, bih_ref, bhh_ref,
                cw1T_ref, cb1_ref, cw2T_ref, cb2_ref,
                rwT_ref, rb_ref, tw1T_ref, tb1_ref, tw2T_ref, tb2_ref,
                ew_ref, ti_ref, rw_ref, comp_ref, hid_ref, lbl_ref, creg_ref,
                h_scr, gi_scr, g_scr, usage_scr, csq_scr):
    i = pl.program_id(0)

    @pl.when(i == 0)
    def _init():
        h_scr[...] = jnp.zeros_like(h_scr)
        usage_scr[...] = jnp.zeros_like(usage_scr)
        csq_scr[...] = jnp.zeros_like(csq_scr)

    xb = x_ref[...].reshape(B * T, D)
    gi = jnp.dot(xb, wxT_ref[...], precision=_PREC,
                 preferred_element_type=jnp.float32) + bih_ref[...]
    gi_scr[...] = gi.reshape(B, T, 3 * H)

    whhT = whhT_ref[...]
    bhh = bhh_ref[...]

    def step(t, h):
        gh = jnp.dot(h, whhT, precision=_PREC,
                     preferred_element_type=jnp.float32) + bhh
        gs = gi_scr[:, t, :]
        r = jax.nn.sigmoid(gs[:, :H] + gh[:, :H])
        z = jax.nn.sigmoid(gs[:, H:2 * H] + gh[:, H:2 * H])
        n = jnp.tanh(gs[:, 2 * H:] + r * gh[:, 2 * H:])
        h2 = (1.0 - z) * n + z * h
        g_scr[:, t, :] = h2
        return h2

    h_last = lax.fori_loop(0, T, step, h_scr[...])
    h_scr[...] = h_last

    g = g_scr[...].reshape(B * T, H)
    c1 = jax.nn.relu(jnp.dot(g, cw1T_ref[...], precision=_PREC,
                             preferred_element_type=jnp.float32) + cb1_ref[...])
    comp = jax.nn.sigmoid(jnp.dot(c1, cw2T_ref[...], precision=_PREC,
                                  preferred_element_type=jnp.float32) + cb2_ref[...])
    logits = jnp.dot(g, rwT_ref[...], precision=_PREC,
                     preferred_element_type=jnp.float32) + rb_ref[...]
    t1 = jax.nn.relu(jnp.dot(g, tw1T_ref[...], precision=_PREC,
                             preferred_element_type=jnp.float32) + tb1_ref[...])
    tks = jax.nn.sigmoid(jnp.dot(t1, tw2T_ref[...], precision=_PREC,
                                 preferred_element_type=jnp.float32) + tb2_ref[...])

    combined = 0.7 * comp + 0.3 * tks
    dyn_k = jnp.floor(1.0 + combined * (MAXK - 1) + 0.5).astype(jnp.int32)

    mx = jnp.max(logits, axis=-1, keepdims=True)
    ex = jnp.exp(logits - mx)
    rw = ex / jnp.sum(ex, axis=-1, keepdims=True)

    iota8 = lax.broadcasted_iota(jnp.int32, (B * T, E), 1)
    cur = rw
    tws = []
    tis = []
    for _ in range(MAXK):
        m = jnp.max(cur, axis=-1, keepdims=True)
        idx = jnp.min(jnp.where(cur >= m, iota8, E), axis=-1, keepdims=True)
        tws.append(m)
        tis.append(idx)
        cur = jnp.where(iota8 == idx, -1.0, cur)
    top_w = jnp.concatenate(tws, axis=-1)
    top_i = jnp.concatenate(tis, axis=-1)

    karange = lax.broadcasted_iota(jnp.int32, (B * T, MAXK), 1)
    kmask = (karange < dyn_k).astype(jnp.float32)
    masked = top_w * kmask
    wsum = jnp.sum(masked, axis=-1, keepdims=True)
    wsum = jnp.where(wsum > 0, wsum, jnp.ones_like(wsum))

    ew_ref[...] = (masked / wsum).reshape(B, T, MAXK)
    ti_ref[...] = top_i.reshape(B, T, MAXK)
    rw_ref[...] = rw.reshape(B, T, E)
    comp_ref[...] = comp.reshape(B, T, 1)

    usage_scr[...] += jnp.sum(rw, axis=0).reshape(1, E)
    csq_scr[...] += jnp.sum(comp * comp).reshape(1, 1)

    @pl.when(i == NBLK - 1)
    def _finish():
        hid_ref[...] = h_scr[...]
        usage = usage_scr[...] / (B * S)
        lbl_ref[...] = (jnp.sum((usage - 1.0 / E) ** 2) / E * 0.01).reshape(1, 1)
        creg_ref[...] = (csq_scr[...] / (B * S) * 0.001).reshape(1, 1)


@functools.partial(jax.jit, static_argnames=("interpret",))
def _router(x, W_ih, W_hh, b_ih, b_hh, ce_w1, ce_b1, ce_w2, ce_b2,
            rh_w, rh_b, tk_w1, tk_b1, tk_w2, tk_b2, interpret=False):
    wxT = W_ih[:, :D].T
    whhT = W_hh.T
    args = (
        x, wxT, whhT,
        b_ih.reshape(1, 3 * H), b_hh.reshape(1, 3 * H),
        ce_w1.T, ce_b1.reshape(1, H // 2), ce_w2.T, ce_b2.reshape(1, 1),
        rh_w.T, rh_b.reshape(1, E),
        tk_w1.T, tk_b1.reshape(1, H // 4), tk_w2.T, tk_b2.reshape(1, 1),
    )
    full = lambda shape: pl.BlockSpec(shape, lambda i: (0,) * len(shape))
    in_specs = [
        pl.BlockSpec((B, T, D), lambda i: (0, i, 0)),
        full((D, 3 * H)), full((H, 3 * H)),
        full((1, 3 * H)), full((1, 3 * H)),
        full((H, H // 2)), full((1, H // 2)), full((H // 2, 1)), full((1, 1)),
        full((H, E)), full((1, E)),
        full((H, H // 4)), full((1, H // 4)), full((H // 4, 1)), full((1, 1)),
    ]
    out_specs = [
        pl.BlockSpec((B, T, MAXK), lambda i: (0, i, 0)),
        pl.BlockSpec((B, T, MAXK), lambda i: (0, i, 0)),
        pl.BlockSpec((B, T, E), lambda i: (0, i, 0)),
        pl.BlockSpec((B, T, 1), lambda i: (0, i, 0)),
        pl.BlockSpec((B, H), lambda i: (0, 0)),
        pl.BlockSpec((1, 1), lambda i: (0, 0)),
        pl.BlockSpec((1, 1), lambda i: (0, 0)),
    ]
    out_shape = [
        jax.ShapeDtypeStruct((B, S, MAXK), jnp.float32),
        jax.ShapeDtypeStruct((B, S, MAXK), jnp.int32),
        jax.ShapeDtypeStruct((B, S, E), jnp.float32),
        jax.ShapeDtypeStruct((B, S, 1), jnp.float32),
        jax.ShapeDtypeStruct((B, H), jnp.float32),
        jax.ShapeDtypeStruct((1, 1), jnp.float32),
        jax.ShapeDtypeStruct((1, 1), jnp.float32),
    ]
    scratch_shapes = [
        pltpu.VMEM((B, H), jnp.float32),
        pltpu.VMEM((B, T, 3 * H), jnp.float32),
        pltpu.VMEM((B, T, H), jnp.float32),
        pltpu.VMEM((1, E), jnp.float32),
        pltpu.VMEM((1, 1), jnp.float32),
    ]
    return pl.pallas_call(
        _fused_body,
        grid=(NBLK,),
        in_specs=in_specs,
        out_specs=out_specs,
        out_shape=out_shape,
        scratch_shapes=scratch_shapes,
        compiler_params=pltpu.CompilerParams(
            dimension_semantics=("arbitrary",),
        ),
        interpret=interpret,
    )(*args)


def kernel(x, layer_idx, W_ih, W_hh, b_ih, b_hh, ce_w1, ce_b1, ce_w2, ce_b2,
           rh_w, rh_b, tk_w1, tk_b1, tk_w2, tk_b2):
    ew, ti, rw, comp, hid, lbl, creg = _router(
        x, W_ih, W_hh, b_ih, b_hh, ce_w1, ce_b1, ce_w2, ce_b2,
        rh_w, rh_b, tk_w1, tk_b1, tk_w2, tk_b2)
    return (ew, ti, rw, comp, hid,
            lbl.reshape(()), creg.reshape(()))


# speedup vs baseline: 8.7616x; 1.1168x over previous
import functools

import jax
import jax.numpy as jnp
from jax import lax
from jax.experimental import pallas as pl
from jax.experimental.pallas import tpu as pltpu

B, S, D = 4, 2048, 1024
E = 8
H = 256
MAXK = 4
T = 128
NBLK = S // T

_PREC = lax.Precision.DEFAULT


def _fused_body(x_ref, wxT_ref, whhT_ref, bih_ref, bhh_ref,
                cw1T_ref, cb1_ref, cw2T_ref, cb2_ref,
                rwT_ref, rb_ref, tw1T_ref, tb1_ref, tw2T_ref, tb2_ref,
                ew_ref, ti_ref, rw_ref, comp_ref, hid_ref, lbl_ref, creg_ref,
                h_scr, gi_scr, g_scr, usage_scr, csq_scr):
    i = pl.program_id(0)

    @pl.when(i == 0)
    def _init():
        h_scr[...] = jnp.zeros_like(h_scr)
        usage_scr[...] = jnp.zeros_like(usage_scr)
        csq_scr[...] = jnp.zeros_like(csq_scr)

    xb = x_ref[...].reshape(B * T, D)
    gi = jnp.dot(xb, wxT_ref[...], precision=_PREC,
                 preferred_element_type=jnp.float32) + bih_ref[...]
    gi_scr[...] = gi.reshape(B, T, 3 * H)

    whhT = whhT_ref[...]
    bhh = bhh_ref[...]

    def step(t, h):
        gh = jnp.dot(h, whhT, precision=_PREC,
                     preferred_element_type=jnp.float32) + bhh
        gs = gi_scr[:, t, :]
        r = jax.nn.sigmoid(gs[:, :H] + gh[:, :H])
        z = jax.nn.sigmoid(gs[:, H:2 * H] + gh[:, H:2 * H])
        n = jnp.tanh(gs[:, 2 * H:] + r * gh[:, 2 * H:])
        h2 = (1.0 - z) * n + z * h
        g_scr[:, t, :] = h2
        return h2

    h_last = lax.fori_loop(0, T, step, h_scr[...], unroll=4)
    h_scr[...] = h_last

    g = g_scr[...].reshape(B * T, H)
    c1 = jax.nn.relu(jnp.dot(g, cw1T_ref[...], precision=_PREC,
                             preferred_element_type=jnp.float32) + cb1_ref[...])
    comp = jax.nn.sigmoid(jnp.dot(c1, cw2T_ref[...], precision=_PREC,
                                  preferred_element_type=jnp.float32) + cb2_ref[...])
    logits = jnp.dot(g, rwT_ref[...], precision=_PREC,
                     preferred_element_type=jnp.float32) + rb_ref[...]
    t1 = jax.nn.relu(jnp.dot(g, tw1T_ref[...], precision=_PREC,
                             preferred_element_type=jnp.float32) + tb1_ref[...])
    tks = jax.nn.sigmoid(jnp.dot(t1, tw2T_ref[...], precision=_PREC,
                                 preferred_element_type=jnp.float32) + tb2_ref[...])

    combined = 0.7 * comp + 0.3 * tks
    dyn_k = jnp.floor(1.0 + combined * (MAXK - 1) + 0.5).astype(jnp.int32)

    mx = jnp.max(logits, axis=-1, keepdims=True)
    ex = jnp.exp(logits - mx)
    rw = ex / jnp.sum(ex, axis=-1, keepdims=True)

    iota8 = lax.broadcasted_iota(jnp.int32, (B * T, E), 1)
    cur = rw
    tws = []
    tis = []
    for _ in range(MAXK):
        m = jnp.max(cur, axis=-1, keepdims=True)
        idx = jnp.min(jnp.where(cur >= m, iota8, E), axis=-1, keepdims=True)
        tws.append(m)
        tis.append(idx)
        cur = jnp.where(iota8 == idx, -1.0, cur)
    top_w = jnp.concatenate(tws, axis=-1)
    top_i = jnp.concatenate(tis, axis=-1)

    karange = lax.broadcasted_iota(jnp.int32, (B * T, MAXK), 1)
    kmask = (karange < dyn_k).astype(jnp.float32)
    masked = top_w * kmask
    wsum = jnp.sum(masked, axis=-1, keepdims=True)
    wsum = jnp.where(wsum > 0, wsum, jnp.ones_like(wsum))

    ew_ref[...] = (masked / wsum).reshape(B, T, MAXK)
    ti_ref[...] = top_i.reshape(B, T, MAXK)
    rw_ref[...] = rw.reshape(B, T, E)
    comp_ref[...] = comp.reshape(B, T, 1)

    usage_scr[...] += jnp.sum(rw, axis=0).reshape(1, E)
    csq_scr[...] += jnp.sum(comp * comp).reshape(1, 1)

    @pl.when(i == NBLK - 1)
    def _finish():
        hid_ref[...] = h_scr[...]
        usage = usage_scr[...] / (B * S)
        lbl_ref[...] = (jnp.sum((usage - 1.0 / E) ** 2) / E * 0.01).reshape(1, 1)
        creg_ref[...] = (csq_scr[...] / (B * S) * 0.001).reshape(1, 1)


@functools.partial(jax.jit, static_argnames=("interpret",))
def _router(x, W_ih, W_hh, b_ih, b_hh, ce_w1, ce_b1, ce_w2, ce_b2,
            rh_w, rh_b, tk_w1, tk_b1, tk_w2, tk_b2, interpret=False):
    wxT = W_ih[:, :D].T
    whhT = W_hh.T
    args = (
        x, wxT, whhT,
        b_ih.reshape(1, 3 * H), b_hh.reshape(1, 3 * H),
        ce_w1.T, ce_b1.reshape(1, H // 2), ce_w2.T, ce_b2.reshape(1, 1),
        rh_w.T, rh_b.reshape(1, E),
        tk_w1.T, tk_b1.reshape(1, H // 4), tk_w2.T, tk_b2.reshape(1, 1),
    )
    full = lambda shape: pl.BlockSpec(shape, lambda i: (0,) * len(shape))
    in_specs = [
        pl.BlockSpec((B, T, D), lambda i: (0, i, 0)),
        full((D, 3 * H)), full((H, 3 * H)),
        full((1, 3 * H)), full((1, 3 * H)),
        full((H, H // 2)), full((1, H // 2)), full((H // 2, 1)), full((1, 1)),
        full((H, E)), full((1, E)),
        full((H, H // 4)), full((1, H // 4)), full((H // 4, 1)), full((1, 1)),
    ]
    out_specs = [
        pl.BlockSpec((B, T, MAXK), lambda i: (0, i, 0)),
        pl.BlockSpec((B, T, MAXK), lambda i: (0, i, 0)),
        pl.BlockSpec((B, T, E), lambda i: (0, i, 0)),
        pl.BlockSpec((B, T, 1), lambda i: (0, i, 0)),
        pl.BlockSpec((B, H), lambda i: (0, 0)),
        pl.BlockSpec((1, 1), lambda i: (0, 0)),
        pl.BlockSpec((1, 1), lambda i: (0, 0)),
    ]
    out_shape = [
        jax.ShapeDtypeStruct((B, S, MAXK), jnp.float32),
        jax.ShapeDtypeStruct((B, S, MAXK), jnp.int32),
        jax.ShapeDtypeStruct((B, S, E), jnp.float32),
        jax.ShapeDtypeStruct((B, S, 1), jnp.float32),
        jax.ShapeDtypeStruct((B, H), jnp.float32),
        jax.ShapeDtypeStruct((1, 1), jnp.float32),
        jax.ShapeDtypeStruct((1, 1), jnp.float32),
    ]
    scratch_shapes = [
        pltpu.VMEM((B, H), jnp.float32),
        pltpu.VMEM((B, T, 3 * H), jnp.float32),
        pltpu.VMEM((B, T, H), jnp.float32),
        pltpu.VMEM((1, E), jnp.float32),
        pltpu.VMEM((1, 1), jnp.float32),
    ]
    return pl.pallas_call(
        _fused_body,
        grid=(NBLK,),
        in_specs=in_specs,
        out_specs=out_specs,
        out_shape=out_shape,
        scratch_shapes=scratch_shapes,
        compiler_params=pltpu.CompilerParams(
            dimension_semantics=("arbitrary",),
        ),
        interpret=interpret,
    )(*args)


def kernel(x, layer_idx, W_ih, W_hh, b_ih, b_hh, ce_w1, ce_b1, ce_w2, ce_b2,
           rh_w, rh_b, tk_w1, tk_b1, tk_w2, tk_b2):
    ew, ti, rw, comp, hid, lbl, creg = _router(
        x, W_ih, W_hh, b_ih, b_hh, ce_w1, ce_b1, ce_w2, ce_b2,
        rh_w, rh_b, tk_w1, tk_b1, tk_w2, tk_b2)
    return (ew, ti, rw, comp, hid,
            lbl.reshape(()), creg.reshape(()))


# speedup vs baseline: 9.3268x; 1.0645x over previous
import functools

import jax
import jax.numpy as jnp
from jax import lax
from jax.experimental import pallas as pl
from jax.experimental.pallas import tpu as pltpu

B, S, D = 4, 2048, 1024
E = 8
H = 256
MAXK = 4
T = 256
NBLK = S // T

_PREC = lax.Precision.DEFAULT


def _fused_body(x_ref, wxT_ref, whhT_ref, bih_ref, bhh_ref,
                cw1T_ref, cb1_ref, cw2T_ref, cb2_ref,
                rwT_ref, rb_ref, tw1T_ref, tb1_ref, tw2T_ref, tb2_ref,
                ew_ref, ti_ref, rw_ref, comp_ref, hid_ref, lbl_ref, creg_ref,
                h_scr, gi_scr, g_scr, usage_scr, csq_scr):
    i = pl.program_id(0)

    @pl.when(i == 0)
    def _init():
        h_scr[...] = jnp.zeros_like(h_scr)
        usage_scr[...] = jnp.zeros_like(usage_scr)
        csq_scr[...] = jnp.zeros_like(csq_scr)

    xb = x_ref[...].reshape(B * T, D)
    gi = jnp.dot(xb, wxT_ref[...], precision=_PREC,
                 preferred_element_type=jnp.float32) + bih_ref[...]
    gi_scr[...] = gi.reshape(B, T, 3 * H)

    def step(t, h):
        gh = jnp.dot(h, whhT_ref[...], precision=_PREC,
                     preferred_element_type=jnp.float32) + bhh_ref[...]
        gs = gi_scr[:, t, :]
        r = jax.nn.sigmoid(gs[:, :H] + gh[:, :H])
        z = jax.nn.sigmoid(gs[:, H:2 * H] + gh[:, H:2 * H])
        n = jnp.tanh(gs[:, 2 * H:] + r * gh[:, 2 * H:])
        h2 = (1.0 - z) * n + z * h
        g_scr[:, t, :] = h2
        return h2

    h_last = lax.fori_loop(0, T, step, h_scr[...], unroll=8)
    h_scr[...] = h_last

    g = g_scr[...].reshape(B * T, H)
    c1 = jax.nn.relu(jnp.dot(g, cw1T_ref[...], precision=_PREC,
                             preferred_element_type=jnp.float32) + cb1_ref[...])
    comp = jax.nn.sigmoid(jnp.dot(c1, cw2T_ref[...], precision=_PREC,
                                  preferred_element_type=jnp.float32) + cb2_ref[...])
    logits = jnp.dot(g, rwT_ref[...], precision=_PREC,
                     preferred_element_type=jnp.float32) + rb_ref[...]
    t1 = jax.nn.relu(jnp.dot(g, tw1T_ref[...], precision=_PREC,
                             preferred_element_type=jnp.float32) + tb1_ref[...])
    tks = jax.nn.sigmoid(jnp.dot(t1, tw2T_ref[...], precision=_PREC,
                                 preferred_element_type=jnp.float32) + tb2_ref[...])

    combined = 0.7 * comp + 0.3 * tks
    dyn_k = jnp.floor(1.0 + combined * (MAXK - 1) + 0.5).astype(jnp.int32)

    mx = jnp.max(logits, axis=-1, keepdims=True)
    ex = jnp.exp(logits - mx)
    rw = ex / jnp.sum(ex, axis=-1, keepdims=True)

    iota8 = lax.broadcasted_iota(jnp.int32, (B * T, E), 1)
    cur = rw
    tws = []
    tis = []
    for _ in range(MAXK):
        m = jnp.max(cur, axis=-1, keepdims=True)
        idx = jnp.min(jnp.where(cur >= m, iota8, E), axis=-1, keepdims=True)
        tws.append(m)
        tis.append(idx)
        cur = jnp.where(iota8 == idx, -1.0, cur)
    top_w = jnp.concatenate(tws, axis=-1)
    top_i = jnp.concatenate(tis, axis=-1)

    karange = lax.broadcasted_iota(jnp.int32, (B * T, MAXK), 1)
    kmask = (karange < dyn_k).astype(jnp.float32)
    masked = top_w * kmask
    wsum = jnp.sum(masked, axis=-1, keepdims=True)
    wsum = jnp.where(wsum > 0, wsum, jnp.ones_like(wsum))

    ew_ref[...] = (masked / wsum).reshape(B, T, MAXK)
    ti_ref[...] = top_i.reshape(B, T, MAXK)
    rw_ref[...] = rw.reshape(B, T, E)
    comp_ref[...] = comp.reshape(B, T, 1)

    usage_scr[...] += jnp.sum(rw, axis=0).reshape(1, E)
    csq_scr[...] += jnp.sum(comp * comp).reshape(1, 1)

    @pl.when(i == NBLK - 1)
    def _finish():
        hid_ref[...] = h_scr[...]
        usage = usage_scr[...] / (B * S)
        lbl_ref[...] = (jnp.sum((usage - 1.0 / E) ** 2) / E * 0.01).reshape(1, 1)
        creg_ref[...] = (csq_scr[...] / (B * S) * 0.001).reshape(1, 1)


@functools.partial(jax.jit, static_argnames=("interpret",))
def _router(x, W_ih, W_hh, b_ih, b_hh, ce_w1, ce_b1, ce_w2, ce_b2,
            rh_w, rh_b, tk_w1, tk_b1, tk_w2, tk_b2, interpret=False):
    wxT = W_ih[:, :D].T
    whhT = W_hh.T
    args = (
        x, wxT, whhT,
        b_ih.reshape(1, 3 * H), b_hh.reshape(1, 3 * H),
        ce_w1.T, ce_b1.reshape(1, H // 2), ce_w2.T, ce_b2.reshape(1, 1),
        rh_w.T, rh_b.reshape(1, E),
        tk_w1.T, tk_b1.reshape(1, H // 4), tk_w2.T, tk_b2.reshape(1, 1),
    )
    full = lambda shape: pl.BlockSpec(shape, lambda i: (0,) * len(shape))
    in_specs = [
        pl.BlockSpec((B, T, D), lambda i: (0, i, 0)),
        full((D, 3 * H)), full((H, 3 * H)),
        full((1, 3 * H)), full((1, 3 * H)),
        full((H, H // 2)), full((1, H // 2)), full((H // 2, 1)), full((1, 1)),
        full((H, E)), full((1, E)),
        full((H, H // 4)), full((1, H // 4)), full((H // 4, 1)), full((1, 1)),
    ]
    out_specs = [
        pl.BlockSpec((B, T, MAXK), lambda i: (0, i, 0)),
        pl.BlockSpec((B, T, MAXK), lambda i: (0, i, 0)),
        pl.BlockSpec((B, T, E), lambda i: (0, i, 0)),
        pl.BlockSpec((B, T, 1), lambda i: (0, i, 0)),
        pl.BlockSpec((B, H), lambda i: (0, 0)),
        pl.BlockSpec((1, 1), lambda i: (0, 0)),
        pl.BlockSpec((1, 1), lambda i: (0, 0)),
    ]
    out_shape = [
        jax.ShapeDtypeStruct((B, S, MAXK), jnp.float32),
        jax.ShapeDtypeStruct((B, S, MAXK), jnp.int32),
        jax.ShapeDtypeStruct((B, S, E), jnp.float32),
        jax.ShapeDtypeStruct((B, S, 1), jnp.float32),
        jax.ShapeDtypeStruct((B, H), jnp.float32),
        jax.ShapeDtypeStruct((1, 1), jnp.float32),
        jax.ShapeDtypeStruct((1, 1), jnp.float32),
    ]
    scratch_shapes = [
        pltpu.VMEM((B, H), jnp.float32),
        pltpu.VMEM((B, T, 3 * H), jnp.float32),
        pltpu.VMEM((B, T, H), jnp.float32),
        pltpu.VMEM((1, E), jnp.float32),
        pltpu.VMEM((1, 1), jnp.float32),
    ]
    return pl.pallas_call(
        _fused_body,
        grid=(NBLK,),
        in_specs=in_specs,
        out_specs=out_specs,
        out_shape=out_shape,
        scratch_shapes=scratch_shapes,
        compiler_params=pltpu.CompilerParams(
            dimension_semantics=("arbitrary",),
        ),
        interpret=interpret,
    )(*args)


def kernel(x, layer_idx, W_ih, W_hh, b_ih, b_hh, ce_w1, ce_b1, ce_w2, ce_b2,
           rh_w, rh_b, tk_w1, tk_b1, tk_w2, tk_b2):
    ew, ti, rw, comp, hid, lbl, creg = _router(
        x, W_ih, W_hh, b_ih, b_hh, ce_w1, ce_b1, ce_w2, ce_b2,
        rh_w, rh_b, tk_w1, tk_b1, tk_w2, tk_b2)
    return (ew, ti, rw, comp, hid,
            lbl.reshape(()), creg.reshape(()))


# speedup vs baseline: 9.3661x; 1.0042x over previous
import functools

import jax
import jax.numpy as jnp
from jax import lax
from jax.experimental import pallas as pl
from jax.experimental.pallas import tpu as pltpu

B, S, D = 4, 2048, 1024
E = 8
H = 256
MAXK = 4
T = 512
NBLK = S // T

_PREC = lax.Precision.DEFAULT


def _fused_body(x_ref, wxT_ref, whhT_ref, bih_ref, bhh_ref,
                cw1T_ref, cb1_ref, cw2T_ref, cb2_ref,
                rwT_ref, rb_ref, tw1T_ref, tb1_ref, tw2T_ref, tb2_ref,
                ew_ref, ti_ref, rw_ref, comp_ref, hid_ref, lbl_ref, creg_ref,
                h_scr, gi_scr, g_scr, usage_scr, csq_scr):
    i = pl.program_id(0)

    @pl.when(i == 0)
    def _init():
        h_scr[...] = jnp.zeros_like(h_scr)
        usage_scr[...] = jnp.zeros_like(usage_scr)
        csq_scr[...] = jnp.zeros_like(csq_scr)

    xb = x_ref[...].reshape(B * T, D)
    gi = jnp.dot(xb, wxT_ref[...], precision=_PREC,
                 preferred_element_type=jnp.float32) + bih_ref[...]
    gi_scr[...] = gi.reshape(B, T, 3 * H)

    def step(t, h):
        gh = jnp.dot(h, whhT_ref[...], precision=_PREC,
                     preferred_element_type=jnp.float32) + bhh_ref[...]
        gs = gi_scr[:, t, :]
        r = jax.nn.sigmoid(gs[:, :H] + gh[:, :H])
        z = jax.nn.sigmoid(gs[:, H:2 * H] + gh[:, H:2 * H])
        n = jnp.tanh(gs[:, 2 * H:] + r * gh[:, 2 * H:])
        h2 = (1.0 - z) * n + z * h
        g_scr[:, t, :] = h2
        return h2

    h_last = lax.fori_loop(0, T, step, h_scr[...], unroll=8)
    h_scr[...] = h_last

    g = g_scr[...].reshape(B * T, H)
    c1 = jax.nn.relu(jnp.dot(g, cw1T_ref[...], precision=_PREC,
                             preferred_element_type=jnp.float32) + cb1_ref[...])
    comp = jax.nn.sigmoid(jnp.dot(c1, cw2T_ref[...], precision=_PREC,
                                  preferred_element_type=jnp.float32) + cb2_ref[...])
    logits = jnp.dot(g, rwT_ref[...], precision=_PREC,
                     preferred_element_type=jnp.float32) + rb_ref[...]
    t1 = jax.nn.relu(jnp.dot(g, tw1T_ref[...], precision=_PREC,
                             preferred_element_type=jnp.float32) + tb1_ref[...])
    tks = jax.nn.sigmoid(jnp.dot(t1, tw2T_ref[...], precision=_PREC,
                                 preferred_element_type=jnp.float32) + tb2_ref[...])

    combined = 0.7 * comp + 0.3 * tks
    dyn_k = jnp.floor(1.0 + combined * (MAXK - 1) + 0.5).astype(jnp.int32)

    mx = jnp.max(logits, axis=-1, keepdims=True)
    ex = jnp.exp(logits - mx)
    rw = ex / jnp.sum(ex, axis=-1, keepdims=True)

    iota8 = lax.broadcasted_iota(jnp.int32, (B * T, E), 1)
    cur = rw
    tws = []
    tis = []
    for _ in range(MAXK):
        m = jnp.max(cur, axis=-1, keepdims=True)
        idx = jnp.min(jnp.where(cur >= m, iota8, E), axis=-1, keepdims=True)
        tws.append(m)
        tis.append(idx)
        cur = jnp.where(iota8 == idx, -1.0, cur)
    top_w = jnp.concatenate(tws, axis=-1)
    top_i = jnp.concatenate(tis, axis=-1)

    karange = lax.broadcasted_iota(jnp.int32, (B * T, MAXK), 1)
    kmask = (karange < dyn_k).astype(jnp.float32)
    masked = top_w * kmask
    wsum = jnp.sum(masked, axis=-1, keepdims=True)
    wsum = jnp.where(wsum > 0, wsum, jnp.ones_like(wsum))

    ew_ref[...] = (masked / wsum).reshape(B, T, MAXK)
    ti_ref[...] = top_i.reshape(B, T, MAXK)
    rw_ref[...] = rw.reshape(B, T, E)
    comp_ref[...] = comp.reshape(B, T, 1)

    usage_scr[...] += jnp.sum(rw, axis=0).reshape(1, E)
    csq_scr[...] += jnp.sum(comp * comp).reshape(1, 1)

    @pl.when(i == NBLK - 1)
    def _finish():
        hid_ref[...] = h_scr[...]
        usage = usage_scr[...] / (B * S)
        lbl_ref[...] = (jnp.sum((usage - 1.0 / E) ** 2) / E * 0.01).reshape(1, 1)
        creg_ref[...] = (csq_scr[...] / (B * S) * 0.001).reshape(1, 1)


@functools.partial(jax.jit, static_argnames=("interpret",))
def _router(x, W_ih, W_hh, b_ih, b_hh, ce_w1, ce_b1, ce_w2, ce_b2,
            rh_w, rh_b, tk_w1, tk_b1, tk_w2, tk_b2, interpret=False):
    wxT = W_ih[:, :D].T
    whhT = W_hh.T
    args = (
        x, wxT, whhT,
        b_ih.reshape(1, 3 * H), b_hh.reshape(1, 3 * H),
        ce_w1.T, ce_b1.reshape(1, H // 2), ce_w2.T, ce_b2.reshape(1, 1),
        rh_w.T, rh_b.reshape(1, E),
        tk_w1.T, tk_b1.reshape(1, H // 4), tk_w2.T, tk_b2.reshape(1, 1),
    )
    full = lambda shape: pl.BlockSpec(shape, lambda i: (0,) * len(shape))
    in_specs = [
        pl.BlockSpec((B, T, D), lambda i: (0, i, 0)),
        full((D, 3 * H)), full((H, 3 * H)),
        full((1, 3 * H)), full((1, 3 * H)),
        full((H, H // 2)), full((1, H // 2)), full((H // 2, 1)), full((1, 1)),
        full((H, E)), full((1, E)),
        full((H, H // 4)), full((1, H // 4)), full((H // 4, 1)), full((1, 1)),
    ]
    out_specs = [
        pl.BlockSpec((B, T, MAXK), lambda i: (0, i, 0)),
        pl.BlockSpec((B, T, MAXK), lambda i: (0, i, 0)),
        pl.BlockSpec((B, T, E), lambda i: (0, i, 0)),
        pl.BlockSpec((B, T, 1), lambda i: (0, i, 0)),
        pl.BlockSpec((B, H), lambda i: (0, 0)),
        pl.BlockSpec((1, 1), lambda i: (0, 0)),
        pl.BlockSpec((1, 1), lambda i: (0, 0)),
    ]
    out_shape = [
        jax.ShapeDtypeStruct((B, S, MAXK), jnp.float32),
        jax.ShapeDtypeStruct((B, S, MAXK), jnp.int32),
        jax.ShapeDtypeStruct((B, S, E), jnp.float32),
        jax.ShapeDtypeStruct((B, S, 1), jnp.float32),
        jax.ShapeDtypeStruct((B, H), jnp.float32),
        jax.ShapeDtypeStruct((1, 1), jnp.float32),
        jax.ShapeDtypeStruct((1, 1), jnp.float32),
    ]
    scratch_shapes = [
        pltpu.VMEM((B, H), jnp.float32),
        pltpu.VMEM((B, T, 3 * H), jnp.float32),
        pltpu.VMEM((B, T, H), jnp.float32),
        pltpu.VMEM((1, E), jnp.float32),
        pltpu.VMEM((1, 1), jnp.float32),
    ]
    return pl.pallas_call(
        _fused_body,
        grid=(NBLK,),
        in_specs=in_specs,
        out_specs=out_specs,
        out_shape=out_shape,
        scratch_shapes=scratch_shapes,
        compiler_params=pltpu.CompilerParams(
            dimension_semantics=("arbitrary",),
        ),
        interpret=interpret,
    )(*args)


def kernel(x, layer_idx, W_ih, W_hh, b_ih, b_hh, ce_w1, ce_b1, ce_w2, ce_b2,
           rh_w, rh_b, tk_w1, tk_b1, tk_w2, tk_b2):
    ew, ti, rw, comp, hid, lbl, creg = _router(
        x, W_ih, W_hh, b_ih, b_hh, ce_w1, ce_b1, ce_w2, ce_b2,
        rh_w, rh_b, tk_w1, tk_b1, tk_w2, tk_b2)
    return (ew, ti, rw, comp, hid,
            lbl.reshape(()), creg.reshape(()))
